# baseline (speedup 1.0000x reference)
import functools

import jax
import jax.numpy as jnp
import numpy as np
from jax import lax
from jax.experimental import pallas as pl
from jax.experimental.pallas import tpu as pltpu
from jax.experimental.pallas import tpu_sc as plsc

NUM_H = 64
EMBED = 256
HEADS = 8
POINTS = 4
HEAD_DIM = EMBED // HEADS
BS = 4
NQ = NUM_H * NUM_H
NBQ = BS * NQ
NROWS = NBQ * HEADS
NS = 4 * POINTS * HEADS

QB = 512
GRID = NBQ // QB

SC_CORES = 2
SC_SUBCORES = 16
NW = SC_CORES * SC_SUBCORES
QPW = NBQ // NW
TQ = 16
NCH = QPW // TQ
SAMP = TQ * NS


def _prep_body(q_ref, wval_ref, bval_ref, wcat_ref, bcat_ref, val_ref, pk_ref):
    q = q_ref[:]
    f32 = jnp.float32
    val = jnp.dot(q, wval_ref[:], preferred_element_type=f32) + bval_ref[:]
    for h in range(HEADS):
        vh = val[:, h * HEAD_DIM:(h + 1) * HEAD_DIM].astype(jnp.bfloat16)
        vs = jnp.concatenate([vh[1:], vh[:1]], axis=0)
        val_ref[h] = jnp.concatenate([vh, vs], axis=1)
    m = jnp.dot(q, wcat_ref[:], preferred_element_type=f32) + bcat_ref[:]
    offx = m[:, 0:32]
    offy = m[:, 32:64]
    logits = m[:, 64:96]

    e = jnp.exp(logits)
    gi = lax.broadcasted_iota(jnp.int32, (32, 32), 0) >> 2
    gj = lax.broadcasted_iota(jnp.int32, (32, 32), 1) >> 2
    G = (gi == gj).astype(f32)
    attnw = e / jnp.dot(e, G, preferred_element_type=f32)

    rowid = pl.program_id(0) * QB + lax.broadcasted_iota(jnp.int32, (QB, 1), 0)
    b = rowid >> 12
    rc = rowid & (NQ - 1)
    r = rc >> 6
    c = rc & (NUM_H - 1)
    scale = np.float32(NUM_H / (NUM_H - 1.0))
    bx = c.astype(f32) * scale - 0.5
    by = r.astype(f32) * scale - 0.5

    lane = lax.broadcasted_iota(jnp.int32, (1, 64), 1)
    dy = lane >> 5
    h_lane = (lane & 31) >> 2

    offx2 = jnp.concatenate([offx] * 2, axis=1)
    offy2 = jnp.concatenate([offy] * 2, axis=1)
    attn2 = jnp.concatenate([attnw] * 2, axis=1)

    ix = bx + offx2
    iy = by + offy2
    x0f = jnp.floor(ix)
    y0f = jnp.floor(iy)
    fx = ix - x0f
    fy = iy - y0f
    x0 = jnp.clip(x0f, -2.0, 65.0).astype(jnp.int32)
    yi = jnp.clip(y0f, -2.0, 65.0).astype(jnp.int32) + dy
    vy = ((yi >= 0) & (yi <= NUM_H - 1)).astype(f32)
    ycl = jnp.clip(yi, 0, NUM_H - 1)
    dyf = dy.astype(f32)
    wy = (dyf * fy + (1.0 - dyf) * (1.0 - fy)) * vy * attn2

    x1 = x0 + 1
    xcp = jnp.clip(x0, 0, NUM_H - 2)
    wx0v = (1.0 - fx) * ((x0 >= 0) & (x0 <= NUM_H - 1)).astype(f32)
    wx1v = fx * ((x1 >= 0) & (x1 <= NUM_H - 1)).astype(f32)
    m00 = (x0 == xcp).astype(f32)
    m10 = (x1 == xcp).astype(f32)
    m01 = (x0 == xcp + 1).astype(f32)
    w_pos0 = wy * (wx0v * m00 + wx1v * m10)
    w_pos1 = wy * (wx0v * m01 + wx1v * m00)

    bh = (b << 3) + h_lane
    pk_ref[:, 0:64] = (bh << 12) + (ycl << 6) + xcp
    pk_ref[:, 64:128] = lax.bitcast_convert_type(w_pos0, jnp.int32)
    pk_ref[:, 128:192] = lax.bitcast_convert_type(w_pos1, jnp.int32)
    pk_ref[:, 192:256] = jnp.zeros((QB, 64), jnp.int32)


def _prep_call(qf, W_val, b_val, wcat, bcat, interpret=False):
    full = lambda s: pl.BlockSpec(s, lambda i: (0, 0))
    return pl.pallas_call(
        _prep_body,
        grid=(GRID,),
        in_specs=[
            pl.BlockSpec((QB, EMBED), lambda i: (i, 0)),
            full((EMBED, EMBED)), full((1, EMBED)),
            full((EMBED, 96)), full((1, 96)),
        ],
        out_specs=[
            pl.BlockSpec((HEADS, QB, 64), lambda i: (i // 8, i % 8, 0)),
            pl.BlockSpec((QB, 2 * NS), lambda i: (i, 0)),
        ],
        out_shape=[
            jax.ShapeDtypeStruct((BS * HEADS, NQ, 64), jnp.bfloat16),
            jax.ShapeDtypeStruct((NBQ, 2 * NS), jnp.int32),
        ],
        interpret=interpret,
    )(qf, W_val, b_val, wcat, bcat)


def _splat(vec, lane):
    return lax.gather(
        vec, jnp.zeros((16, 1), jnp.int32) + lane,
        lax.GatherDimensionNumbers(offset_dims=(), collapsed_slice_dims=(0,),
                                   start_index_map=(0,)),
        (1,), mode=lax.GatherScatterMode.PROMISE_IN_BOUNDS)


def _sc_body(table_hbm, pk_hbm, out_hbm, pkA, pkB, rowsA, rowsB, outA, outB,
             semIOA, semIOB, semGA, semGB, semOA, semOB):
    wid = lax.axis_index("s") * SC_CORES + lax.axis_index("c")
    q_base = wid * QPW

    def io_copy(ch, pk_v, sem):
        return pltpu.make_async_copy(
            pk_hbm.at[pl.ds(q_base + ch * TQ, TQ)], pk_v, sem)

    def g_copies(pk_v, rows_v, sem):
        return [
            pltpu.make_async_copy(table_hbm.at[pk_v.at[k, pl.ds(0, 64)]],
                                  rows_v.at[pl.ds(k * 64, 64)], sem)
            for k in range(TQ)
        ]

    def g_drain(rows_v, sem):
        pltpu.make_async_copy(table_hbm.at[pl.ds(0, TQ * 64)], rows_v,
                              sem).wait()

    def o_copy(ch, out_v, sem):
        return pltpu.make_async_copy(
            out_v, out_hbm.at[pl.ds((q_base + ch * TQ) * HEADS, TQ * HEADS)],
            sem)

    def compute(pk_v, rows_v, out_v):
        @plsc.parallel_loop(0, TQ, step=1)
        def q_body(qq):
            base = qq * 64
            for hq in range(2):
                wv = [[
                    plsc.bitcast(
                        pk_v[qq, pl.ds(64 + pos * 64 + dy * 32 + hq * 16, 16)],
                        jnp.float32)
                    for pos in range(2)
                ] for dy in range(2)]
                for h4 in range(4):
                    h = hq * 4 + h4
                    accE = jnp.zeros((16,), jnp.float32)
                    accO = jnp.zeros((16,), jnp.float32)
                    for dy in range(2):
                        for p in range(POINTS):
                            row = base + dy * 32 + h * 4 + p
                            for pos in range(2):
                                spl = _splat(wv[dy][pos], h4 * 4 + p)
                                ev, od = plsc.unpack(
                                    rows_v[row, pl.ds(pos * 32, 32)],
                                    format=plsc.PackFormat.INTERLEAVED)
                                accE = accE + spl * ev
                                accO = accO + spl * od
                    out_v[qq * HEADS + h, :] = plsc.pack(
                        accE, accO, format=plsc.PackFormat.INTERLEAVED)

    c0 = io_copy(0, pkA, semIOA)
    c0.start()
    c0.wait()
    for c in g_copies(pkA, rowsA, semGA):
        c.start()
    io_copy(1, pkB, semIOB).start()

    def pair_body(i, carry):
        k0 = 2 * i

        def half(ch, pk_v, rows_v, out_v, semIO, semG, semO,
                 pk_o, rows_o, semIO_o, semG_o):
            @pl.when(ch + 1 <= NCH - 1)
            def _():
                io_copy(ch + 1, pk_o, semIO_o).wait()
                for c in g_copies(pk_o, rows_o, semG_o):
                    c.start()

            g_drain(rows_v, semG)

            @pl.when(ch >= 2)
            def _():
                o_copy(ch - 2, out_v, semO).wait()

            compute(pk_v, rows_v, out_v)
            o_copy(ch, out_v, semO).start()

            @pl.when(ch + 2 <= NCH - 1)
            def _():
                io_copy(ch + 2, pk_v, semIO).start()

        half(k0, pkA, rowsA, outA, semIOA, semGA, semOA,
             pkB, rowsB, semIOB, semGB)
        half(k0 + 1, pkB, rowsB, outB, semIOB, semGB, semOB,
             pkA, rowsA, semIOA, semGA)
        return carry

    lax.fori_loop(0, NCH // 2, pair_body, 0)

    o_copy(NCH - 2, outA, semOA).wait()
    o_copy(NCH - 1, outB, semOB).wait()


def _sc_call(table, pk):
    mesh = plsc.VectorSubcoreMesh(core_axis_name="c", subcore_axis_name="s")
    return pl.kernel(
        _sc_body,
        out_type=jax.ShapeDtypeStruct((NROWS, HEAD_DIM), jnp.bfloat16),
        mesh=mesh,
        scratch_types=[
            pltpu.VMEM((TQ, 2 * NS), jnp.int32),
            pltpu.VMEM((TQ, 2 * NS), jnp.int32),
            pltpu.VMEM((TQ * 64, 64), jnp.bfloat16),
            pltpu.VMEM((TQ * 64, 64), jnp.bfloat16),
            pltpu.VMEM((TQ * HEADS, HEAD_DIM), jnp.bfloat16),
            pltpu.VMEM((TQ * HEADS, HEAD_DIM), jnp.bfloat16),
            pltpu.SemaphoreType.DMA,
            pltpu.SemaphoreType.DMA,
            pltpu.SemaphoreType.DMA,
            pltpu.SemaphoreType.DMA,
            pltpu.SemaphoreType.DMA,
            pltpu.SemaphoreType.DMA,
        ],
        compiler_params=pltpu.CompilerParams(needs_layout_passes=False,
                                             use_tc_tiling_on_sc=False),
    )(table, pk)


def _out_body(s_ref, q_ref, wout_ref, bout_ref, o_ref):
    o_ref[:] = (jnp.dot(s_ref[:], wout_ref[:], preferred_element_type=jnp.float32)
                + bout_ref[:] + 2.0 * q_ref[:])


def _out_call(smp, qf, W_out, b_out, interpret=False):
    return pl.pallas_call(
        _out_body,
        grid=(GRID,),
        in_specs=[
            pl.BlockSpec((QB, EMBED), lambda i: (i, 0)),
            pl.BlockSpec((QB, EMBED), lambda i: (i, 0)),
            pl.BlockSpec((EMBED, EMBED), lambda i: (0, 0)),
            pl.BlockSpec((1, EMBED), lambda i: (0, 0)),
        ],
        out_specs=pl.BlockSpec((QB, EMBED), lambda i: (i, 0)),
        out_shape=jax.ShapeDtypeStruct((NBQ, EMBED), jnp.float32),
        interpret=interpret,
    )(smp, qf, W_out, b_out)


def kernel(query, W_off, b_off, W_attn, b_attn, W_val, b_val, W_out, b_out):
    qf = query.reshape(NBQ, EMBED)
    wcat = jnp.concatenate([W_off[:, 0::2], W_off[:, 1::2], W_attn], axis=1)
    bcat = jnp.concatenate([b_off[0::2], b_off[1::2], b_attn]).reshape(1, 96)

    val2, pk = _prep_call(qf, W_val, b_val.reshape(1, EMBED), wcat, bcat)
    table = val2.reshape(NROWS, 2 * HEAD_DIM)
    smp = _sc_call(table, pk)
    out = _out_call(smp.reshape(NBQ, EMBED), qf, W_out, b_out.reshape(1, EMBED))
    return out.reshape(BS, NQ, EMBED)


# speedup vs baseline: 1.1827x; 1.1827x over previous
import functools

import jax
import jax.numpy as jnp
import numpy as np
from jax import lax
from jax.experimental import pallas as pl
from jax.experimental.pallas import tpu as pltpu
from jax.experimental.pallas import tpu_sc as plsc

NUM_H = 64
EMBED = 256
HEADS = 8
POINTS = 4
HEAD_DIM = EMBED // HEADS
BS = 4
NQ = NUM_H * NUM_H
NBQ = BS * NQ
NROWS = NBQ * HEADS
NS = 4 * POINTS * HEADS

QB = 512
GRID = NBQ // QB

SC_CORES = 2
SC_SUBCORES = 16
NW = SC_CORES * SC_SUBCORES
QPW = NBQ // NW
TQ = 16
NCH = QPW // TQ
SAMP = TQ * NS


def _prep_body(q_ref, wval_ref, bval_ref, wcat_ref, bcat_ref, val_ref, pk_ref):
    q = q_ref[:]
    f32 = jnp.float32
    val = jnp.dot(q, wval_ref[:], preferred_element_type=f32) + bval_ref[:]
    val_ref[:] = val.astype(jnp.bfloat16)
    m = jnp.dot(q, wcat_ref[:], preferred_element_type=f32) + bcat_ref[:]
    offx = m[:, 0:32]
    offy = m[:, 32:64]
    logits = m[:, 64:96]

    e = jnp.exp(logits)
    gi = lax.broadcasted_iota(jnp.int32, (32, 32), 0) >> 2
    gj = lax.broadcasted_iota(jnp.int32, (32, 32), 1) >> 2
    G = (gi == gj).astype(f32)
    attnw = e / jnp.dot(e, G, preferred_element_type=f32)

    rowid = pl.program_id(0) * QB + lax.broadcasted_iota(jnp.int32, (QB, 1), 0)
    b = rowid >> 12
    rc = rowid & (NQ - 1)
    r = rc >> 6
    c = rc & (NUM_H - 1)
    scale = np.float32(NUM_H / (NUM_H - 1.0))
    bx = c.astype(f32) * scale - 0.5
    by = r.astype(f32) * scale - 0.5

    lane = lax.broadcasted_iota(jnp.int32, (1, NS), 1)
    dx = (lane >> 6) & 1
    dy = (lane >> 5) & 1
    h_lane = (lane & 31) >> 2

    offx4 = jnp.concatenate([offx] * 4, axis=1)
    offy4 = jnp.concatenate([offy] * 4, axis=1)
    attn4 = jnp.concatenate([attnw] * 4, axis=1)

    ix = bx + offx4
    iy = by + offy4
    x0f = jnp.floor(ix)
    y0f = jnp.floor(iy)
    fx = ix - x0f
    fy = iy - y0f
    xi = jnp.clip(x0f, -2.0, 65.0).astype(jnp.int32) + dx
    yi = jnp.clip(y0f, -2.0, 65.0).astype(jnp.int32) + dy
    valid = ((xi >= 0) & (xi <= NUM_H - 1) & (yi >= 0)
             & (yi <= NUM_H - 1)).astype(f32)
    xc = jnp.clip(xi, 0, NUM_H - 1)
    yc = jnp.clip(yi, 0, NUM_H - 1)
    dxf = dx.astype(f32)
    dyf = dy.astype(f32)
    wx = dxf * fx + (1.0 - dxf) * (1.0 - fx)
    wy = dyf * fy + (1.0 - dyf) * (1.0 - fy)

    sb = b << 12
    pk_ref[:, 0:NS] = (((sb + (yc << 6) + xc) << 3) + h_lane)
    pk_ref[:, NS:2 * NS] = lax.bitcast_convert_type(attn4 * wx * wy * valid,
                                                    jnp.int32)


def _prep_call(qf, W_val, b_val, wcat, bcat, interpret=False):
    full = lambda s: pl.BlockSpec(s, lambda i: (0, 0))
    return pl.pallas_call(
        _prep_body,
        grid=(GRID,),
        in_specs=[
            pl.BlockSpec((QB, EMBED), lambda i: (i, 0)),
            full((EMBED, EMBED)), full((1, EMBED)),
            full((EMBED, 96)), full((1, 96)),
        ],
        out_specs=[
            pl.BlockSpec((QB, EMBED), lambda i: (i, 0)),
            pl.BlockSpec((QB, 2 * NS), lambda i: (i, 0)),
        ],
        out_shape=[
            jax.ShapeDtypeStruct((NBQ, EMBED), jnp.bfloat16),
            jax.ShapeDtypeStruct((NBQ, 2 * NS), jnp.int32),
        ],
        interpret=interpret,
    )(qf, W_val, b_val, wcat, bcat)


def _splat(vec, lane):
    return lax.gather(
        vec, jnp.zeros((16, 1), jnp.int32) + lane,
        lax.GatherDimensionNumbers(offset_dims=(), collapsed_slice_dims=(0,),
                                   start_index_map=(0,)),
        (1,), mode=lax.GatherScatterMode.PROMISE_IN_BOUNDS)


def _sc_body(table_hbm, pk_hbm, out_hbm, pkA, pkB, rowsA, rowsB, outA, outB,
             semIOA, semIOB, semGA, semGB, semOA, semOB):
    wid = lax.axis_index("s") * SC_CORES + lax.axis_index("c")
    q_base = wid * QPW

    def io_copy(ch, pk_v, sem):
        return pltpu.make_async_copy(
            pk_hbm.at[pl.ds(q_base + ch * TQ, TQ)], pk_v, sem)

    def g_copies(pk_v, rows_v, sem):
        return [
            pltpu.make_async_copy(table_hbm.at[pk_v.at[k, pl.ds(0, NS)]],
                                  rows_v.at[pl.ds(k * NS, NS)], sem)
            for k in range(TQ)
        ]

    def g_drain(rows_v, sem):
        pltpu.make_async_copy(table_hbm.at[pl.ds(0, SAMP)], rows_v, sem).wait()

    def o_copy(ch, out_v, sem):
        return pltpu.make_async_copy(
            out_v, out_hbm.at[pl.ds((q_base + ch * TQ) * HEADS, TQ * HEADS)],
            sem)

    def compute(pk_v, rows_v, out_v):
        def q_body(qq, carry):
            base = qq * NS
            for hg in range(2):
                w16 = [
                    plsc.bitcast(
                        pk_v[qq, pl.ds(NS + c4 * 32 + hg * 16, 16)],
                        jnp.float32)
                    for c4 in range(4)
                ]
                for h4 in range(4):
                    accE = jnp.zeros((16,), jnp.float32)
                    accO = jnp.zeros((16,), jnp.float32)
                    for c4 in range(4):
                        for p in range(POINTS):
                            lane = h4 * 4 + p
                            spl = _splat(w16[c4], lane)
                            pos = base + c4 * 32 + hg * 16 + lane
                            ev, od = plsc.unpack(
                                rows_v[pos, :],
                                format=plsc.PackFormat.INTERLEAVED)
                            accE = accE + spl * ev
                            accO = accO + spl * od
                    h = hg * 4 + h4
                    out_v[qq * HEADS + h, :] = plsc.pack(
                        accE, accO, format=plsc.PackFormat.INTERLEAVED)
            return carry

        lax.fori_loop(0, TQ, q_body, 0)

    c0 = io_copy(0, pkA, semIOA)
    c0.start()
    c0.wait()
    for c in g_copies(pkA, rowsA, semGA):
        c.start()
    io_copy(1, pkB, semIOB).start()

    def pair_body(i, carry):
        k0 = 2 * i

        def half(ch, pk_v, rows_v, out_v, semIO, semG, semO,
                 pk_o, rows_o, semIO_o, semG_o):
            @pl.when(ch + 1 <= NCH - 1)
            def _():
                io_copy(ch + 1, pk_o, semIO_o).wait()
                for c in g_copies(pk_o, rows_o, semG_o):
                    c.start()

            g_drain(rows_v, semG)

            @pl.when(ch >= 2)
            def _():
                o_copy(ch - 2, out_v, semO).wait()

            compute(pk_v, rows_v, out_v)
            o_copy(ch, out_v, semO).start()

            @pl.when(ch + 2 <= NCH - 1)
            def _():
                io_copy(ch + 2, pk_v, semIO).start()

        half(k0, pkA, rowsA, outA, semIOA, semGA, semOA,
             pkB, rowsB, semIOB, semGB)
        half(k0 + 1, pkB, rowsB, outB, semIOB, semGB, semOB,
             pkA, rowsA, semIOA, semGA)
        return carry

    lax.fori_loop(0, NCH // 2, pair_body, 0)

    o_copy(NCH - 2, outA, semOA).wait()
    o_copy(NCH - 1, outB, semOB).wait()


def _sc_call(table, pk):
    mesh = plsc.VectorSubcoreMesh(core_axis_name="c", subcore_axis_name="s")
    return pl.kernel(
        _sc_body,
        out_type=jax.ShapeDtypeStruct((NROWS, HEAD_DIM), jnp.bfloat16),
        mesh=mesh,
        scratch_types=[
            pltpu.VMEM((TQ, 2 * NS), jnp.int32),
            pltpu.VMEM((TQ, 2 * NS), jnp.int32),
            pltpu.VMEM((SAMP, HEAD_DIM), jnp.bfloat16),
            pltpu.VMEM((SAMP, HEAD_DIM), jnp.bfloat16),
            pltpu.VMEM((TQ * HEADS, HEAD_DIM), jnp.bfloat16),
            pltpu.VMEM((TQ * HEADS, HEAD_DIM), jnp.bfloat16),
            pltpu.SemaphoreType.DMA,
            pltpu.SemaphoreType.DMA,
            pltpu.SemaphoreType.DMA,
            pltpu.SemaphoreType.DMA,
            pltpu.SemaphoreType.DMA,
            pltpu.SemaphoreType.DMA,
        ],
        compiler_params=pltpu.CompilerParams(needs_layout_passes=False,
                                             use_tc_tiling_on_sc=False),
    )(table, pk)


def _out_body(s_ref, q_ref, wout_ref, bout_ref, o_ref):
    o_ref[:] = (jnp.dot(s_ref[:], wout_ref[:], preferred_element_type=jnp.float32)
                + bout_ref[:] + 2.0 * q_ref[:])


def _out_call(smp, qf, W_out, b_out, interpret=False):
    return pl.pallas_call(
        _out_body,
        grid=(GRID,),
        in_specs=[
            pl.BlockSpec((QB, EMBED), lambda i: (i, 0)),
            pl.BlockSpec((QB, EMBED), lambda i: (i, 0)),
            pl.BlockSpec((EMBED, EMBED), lambda i: (0, 0)),
            pl.BlockSpec((1, EMBED), lambda i: (0, 0)),
        ],
        out_specs=pl.BlockSpec((QB, EMBED), lambda i: (i, 0)),
        out_shape=jax.ShapeDtypeStruct((NBQ, EMBED), jnp.float32),
        interpret=interpret,
    )(smp, qf, W_out, b_out)


def kernel(query, W_off, b_off, W_attn, b_attn, W_val, b_val, W_out, b_out):
    qf = query.reshape(NBQ, EMBED)
    wcat = jnp.concatenate([W_off[:, 0::2], W_off[:, 1::2], W_attn], axis=1)
    bcat = jnp.concatenate([b_off[0::2], b_off[1::2], b_attn]).reshape(1, 96)

    val, pk = _prep_call(qf, W_val, b_val.reshape(1, EMBED), wcat, bcat)
    table = val.reshape(NROWS, HEAD_DIM)
    smp = _sc_call(table, pk)
    out = _out_call(smp.reshape(NBQ, EMBED), qf, W_out, b_out.reshape(1, EMBED))
    return out.reshape(BS, NQ, EMBED)


# speedup vs baseline: 1.2003x; 1.0148x over previous
import functools

import jax
import jax.numpy as jnp
import numpy as np
from jax import lax
from jax.experimental import pallas as pl
from jax.experimental.pallas import tpu as pltpu
from jax.experimental.pallas import tpu_sc as plsc

NUM_H = 64
EMBED = 256
HEADS = 8
POINTS = 4
HEAD_DIM = EMBED // HEADS
BS = 4
NQ = NUM_H * NUM_H
NBQ = BS * NQ
NROWS = NBQ * HEADS
NS = 4 * POINTS * HEADS

QB = 512
GRID = NBQ // QB

SC_CORES = 2
SC_SUBCORES = 16
NW = SC_CORES * SC_SUBCORES
QPW = NBQ // NW
TQ = 16
NCH = QPW // TQ
SAMP = TQ * NS


def _prep_body(q_ref, wval_ref, bval_ref, woffx_ref, boffx_ref, woffy_ref,
               boffy_ref, wattn_ref, battn_ref, val_ref, pk_ref):
    q = q_ref[:]
    f32 = jnp.float32
    val = jnp.dot(q, wval_ref[:], preferred_element_type=f32) + bval_ref[:]
    val_ref[:] = val.astype(jnp.bfloat16)
    offx = jnp.dot(q, woffx_ref[:], preferred_element_type=f32) + boffx_ref[:]
    offy = jnp.dot(q, woffy_ref[:], preferred_element_type=f32) + boffy_ref[:]
    logits = jnp.dot(q, wattn_ref[:], preferred_element_type=f32) + battn_ref[:]

    e = jnp.exp(logits)
    gi = lax.broadcasted_iota(jnp.int32, (32, 32), 0) >> 2
    gj = lax.broadcasted_iota(jnp.int32, (32, 32), 1) >> 2
    G = (gi == gj).astype(f32)
    attnw = e / jnp.dot(e, G, preferred_element_type=f32)

    rowid = pl.program_id(0) * QB + lax.broadcasted_iota(jnp.int32, (QB, 1), 0)
    b = rowid >> 12
    rc = rowid & (NQ - 1)
    r = rc >> 6
    c = rc & (NUM_H - 1)
    scale = np.float32(NUM_H / (NUM_H - 1.0))
    ix = c.astype(f32) * scale - 0.5 + offx
    iy = r.astype(f32) * scale - 0.5 + offy

    x0f = jnp.floor(ix)
    y0f = jnp.floor(iy)
    fx = ix - x0f
    fy = iy - y0f
    x0 = jnp.clip(x0f, -2.0, 65.0).astype(jnp.int32)
    y0 = jnp.clip(y0f, -2.0, 65.0).astype(jnp.int32)
    x1 = x0 + 1
    y1 = y0 + 1

    def v(t):
        return ((t >= 0) & (t <= NUM_H - 1)).astype(f32)

    vx0, vx1, vy0, vy1 = v(x0), v(x1), v(y0), v(y1)
    xc0 = jnp.clip(x0, 0, NUM_H - 1)
    xc1 = jnp.clip(x1, 0, NUM_H - 1)
    yc0 = jnp.clip(y0, 0, NUM_H - 1)
    yc1 = jnp.clip(y1, 0, NUM_H - 1)

    h_lane = lax.broadcasted_iota(jnp.int32, (1, 32), 1) >> 2
    sb = b << 12

    def rid(yc, xc):
        return ((sb + (yc << 6) + xc) << 3) + h_lane

    pk_ref[:, 0:32] = rid(yc0, xc0)
    pk_ref[:, 32:64] = rid(yc1, xc0)
    pk_ref[:, 64:96] = rid(yc0, xc1)
    pk_ref[:, 96:128] = rid(yc1, xc1)

    wx0 = 1.0 - fx
    wy0 = 1.0 - fy

    def wbits(w):
        return lax.bitcast_convert_type(w, jnp.int32)

    pk_ref[:, 128:160] = wbits(attnw * wx0 * wy0 * vx0 * vy0)
    pk_ref[:, 160:192] = wbits(attnw * wx0 * fy * vx0 * vy1)
    pk_ref[:, 192:224] = wbits(attnw * fx * wy0 * vx1 * vy0)
    pk_ref[:, 224:256] = wbits(attnw * fx * fy * vx1 * vy1)


def _prep_call(qf, W_val, b_val, woffx, boffx, woffy, boffy, W_attn, b_attn,
               interpret=False):
    full = lambda s: pl.BlockSpec(s, lambda i: (0, 0))
    return pl.pallas_call(
        _prep_body,
        grid=(GRID,),
        in_specs=[
            pl.BlockSpec((QB, EMBED), lambda i: (i, 0)),
            full((EMBED, EMBED)), full((1, EMBED)),
            full((EMBED, 32)), full((1, 32)),
            full((EMBED, 32)), full((1, 32)),
            full((EMBED, 32)), full((1, 32)),
        ],
        out_specs=[
            pl.BlockSpec((QB, EMBED), lambda i: (i, 0)),
            pl.BlockSpec((QB, 2 * NS), lambda i: (i, 0)),
        ],
        out_shape=[
            jax.ShapeDtypeStruct((NBQ, EMBED), jnp.bfloat16),
            jax.ShapeDtypeStruct((NBQ, 2 * NS), jnp.int32),
        ],
        interpret=interpret,
    )(qf, W_val, b_val, woffx, boffx, woffy, boffy, W_attn, b_attn)


def _splat(vec, lane):
    return lax.gather(
        vec, jnp.zeros((16, 1), jnp.int32) + lane,
        lax.GatherDimensionNumbers(offset_dims=(), collapsed_slice_dims=(0,),
                                   start_index_map=(0,)),
        (1,), mode=lax.GatherScatterMode.PROMISE_IN_BOUNDS)


def _sc_body(table_hbm, pk_hbm, out_hbm, pkA, pkB, rowsA, rowsB, outA, outB,
             semIOA, semIOB, semGA, semGB, semOA, semOB):
    wid = lax.axis_index("s") * SC_CORES + lax.axis_index("c")
    q_base = wid * QPW

    def io_copy(ch, pk_v, sem):
        return pltpu.make_async_copy(
            pk_hbm.at[pl.ds(q_base + ch * TQ, TQ)], pk_v, sem)

    def g_copies(pk_v, rows_v, sem):
        return [
            pltpu.make_async_copy(table_hbm.at[pk_v.at[k, pl.ds(0, NS)]],
                                  rows_v.at[pl.ds(k * NS, NS)], sem)
            for k in range(TQ)
        ]

    def o_copy(ch, out_v, sem):
        return pltpu.make_async_copy(
            out_v, out_hbm.at[pl.ds((q_base + ch * TQ) * HEADS, TQ * HEADS)],
            sem)

    def compute(pk_v, rows_v, out_v):
        def q_body(qq, carry2):
            base = qq * NS
            for hg in range(2):
                w16 = [
                    plsc.bitcast(
                        pk_v[qq, pl.ds(NS + c4 * 32 + hg * 16, 16)],
                        jnp.float32)
                    for c4 in range(4)
                ]
                for h4 in range(4):
                    accE = jnp.zeros((16,), jnp.float32)
                    accO = jnp.zeros((16,), jnp.float32)
                    for c4 in range(4):
                        for p in range(POINTS):
                            lane = h4 * 4 + p
                            spl = _splat(w16[c4], lane)
                            pos = base + c4 * 32 + hg * 16 + lane
                            ev, od = plsc.unpack(
                                rows_v[pos, :],
                                format=plsc.PackFormat.INTERLEAVED)
                            accE = accE + spl * ev
                            accO = accO + spl * od
                    h = hg * 4 + h4
                    out_v[qq * HEADS + h, pl.ds(0, 16)] = accE
                    out_v[qq * HEADS + h, pl.ds(16, 16)] = accO
            return carry2

        lax.fori_loop(0, TQ, q_body, 0)

    c0 = io_copy(0, pkA, semIOA)
    c0.start()
    c0.wait()
    for c in g_copies(pkA, rowsA, semGA):
        c.start()
    io_copy(1, pkB, semIOB).start()

    def pair_body(i, carry):
        k0 = 2 * i

        def half(ch, pk_v, rows_v, out_v, semIO, semG, semO,
                 pk_o, rows_o, semIO_o, semG_o):
            @pl.when(ch + 1 <= NCH - 1)
            def _():
                io_copy(ch + 1, pk_o, semIO_o).wait()
                for c in g_copies(pk_o, rows_o, semG_o):
                    c.start()

            for c in g_copies(pk_v, rows_v, semG):
                c.wait()

            @pl.when(ch >= 2)
            def _():
                o_copy(ch - 2, out_v, semO).wait()

            compute(pk_v, rows_v, out_v)
            o_copy(ch, out_v, semO).start()

            @pl.when(ch + 2 <= NCH - 1)
            def _():
                io_copy(ch + 2, pk_v, semIO).start()

        half(k0, pkA, rowsA, outA, semIOA, semGA, semOA,
             pkB, rowsB, semIOB, semGB)
        half(k0 + 1, pkB, rowsB, outB, semIOB, semGB, semOB,
             pkA, rowsA, semIOA, semGA)
        return carry

    lax.fori_loop(0, NCH // 2, pair_body, 0)

    o_copy(NCH - 2, outA, semOA).wait()
    o_copy(NCH - 1, outB, semOB).wait()


def _sc_call(table, pk):
    mesh = plsc.VectorSubcoreMesh(core_axis_name="c", subcore_axis_name="s")
    return pl.kernel(
        _sc_body,
        out_type=jax.ShapeDtypeStruct((NROWS, HEAD_DIM), jnp.float32),
        mesh=mesh,
        scratch_types=[
            pltpu.VMEM((TQ, 2 * NS), jnp.int32),
            pltpu.VMEM((TQ, 2 * NS), jnp.int32),
            pltpu.VMEM((SAMP, HEAD_DIM), jnp.bfloat16),
            pltpu.VMEM((SAMP, HEAD_DIM), jnp.bfloat16),
            pltpu.VMEM((TQ * HEADS, HEAD_DIM), jnp.float32),
            pltpu.VMEM((TQ * HEADS, HEAD_DIM), jnp.float32),
            pltpu.SemaphoreType.DMA,
            pltpu.SemaphoreType.DMA,
            pltpu.SemaphoreType.DMA,
            pltpu.SemaphoreType.DMA,
            pltpu.SemaphoreType.DMA,
            pltpu.SemaphoreType.DMA,
        ],
        compiler_params=pltpu.CompilerParams(needs_layout_passes=False,
                                             use_tc_tiling_on_sc=False),
    )(table, pk)


def _out_body(s_ref, q_ref, wout_ref, bout_ref, o_ref):
    o_ref[:] = (jnp.dot(s_ref[:], wout_ref[:], preferred_element_type=jnp.float32)
                + bout_ref[:] + 2.0 * q_ref[:])


def _out_call(smp, qf, W_out, b_out, interpret=False):
    return pl.pallas_call(
        _out_body,
        grid=(GRID,),
        in_specs=[
            pl.BlockSpec((QB, EMBED), lambda i: (i, 0)),
            pl.BlockSpec((QB, EMBED), lambda i: (i, 0)),
            pl.BlockSpec((EMBED, EMBED), lambda i: (0, 0)),
            pl.BlockSpec((1, EMBED), lambda i: (0, 0)),
        ],
        out_specs=pl.BlockSpec((QB, EMBED), lambda i: (i, 0)),
        out_shape=jax.ShapeDtypeStruct((NBQ, EMBED), jnp.float32),
        interpret=interpret,
    )(smp, qf, W_out, b_out)


_UNPACK_ORDER = np.concatenate([np.arange(0, 32, 2), np.arange(1, 32, 2)])
_WOUT_PERM = np.concatenate([h * 32 + _UNPACK_ORDER for h in range(HEADS)])


def kernel(query, W_off, b_off, W_attn, b_attn, W_val, b_val, W_out, b_out):
    qf = query.reshape(NBQ, EMBED)
    woffx = W_off[:, 0::2]
    woffy = W_off[:, 1::2]
    boffx = b_off[0::2].reshape(1, 32)
    boffy = b_off[1::2].reshape(1, 32)

    val, pk = _prep_call(qf, W_val, b_val.reshape(1, EMBED), woffx, boffx,
                         woffy, boffy, W_attn, b_attn.reshape(1, 32))
    table = val.reshape(NROWS, HEAD_DIM)
    smp = _sc_call(table, pk)
    out = _out_call(smp.reshape(NBQ, EMBED), qf, W_out[_WOUT_PERM, :],
                    b_out.reshape(1, EMBED))
    return out.reshape(BS, NQ, EMBED)
